# Initial kernel scaffold; baseline (speedup 1.0000x reference)
import jax
import jax.numpy as jnp
from jax.experimental import pallas as pl


def kernel(x, edge_index, W_in, b_in, W_h, b_h, W_out, b_out):
    raise NotImplementedError("write your pallas kernel here")


# speedup vs baseline: 12.6384x; 12.6384x over previous
import functools

import jax
import jax.numpy as jnp
from jax import lax
from jax.experimental import pallas as pl
from jax.experimental.pallas import tpu as pltpu
from jax.experimental.pallas import tpu_sc as plsc

N_NODES_K = 10000
D_FEAT_K = 128
HIDDEN_K = 128
N_CLASSES_K = 40
N_EDGES_K = 320000

_info = plsc.get_sparse_core_info()
NC = _info.num_cores
NS = _info.num_subcores
NW = NC * NS

CH = 128
NCH = -(-N_EDGES_K // (NW * CH))
EPW = NCH * CH
EPAD = NW * EPW

NPAD = 10240
ROWS_PER_TILE = NPAD // NS
BN = 1024
GRID = NPAD // BN

D_OUT_PAD = 64

_MESH = plsc.VectorSubcoreMesh(core_axis_name="c", subcore_axis_name="s")
_SC_PARAMS = pltpu.CompilerParams(needs_layout_passes=False)
_SC_PARAMS_LINEAR = pltpu.CompilerParams(
    needs_layout_passes=False, use_tc_tiling_on_sc=False)


@functools.partial(
    pl.kernel,
    mesh=_MESH,
    out_type=jax.ShapeDtypeStruct((NW, NPAD), jnp.float32),
    scratch_types=[
        pltpu.VMEM((EPW,), jnp.int32),
        pltpu.VMEM((NPAD,), jnp.float32),
    ],
    compiler_params=_SC_PARAMS,
)
def _deg_kernel(didx_hbm, degp_hbm, didx_v, dacc_v):
    cc = lax.axis_index("c")
    s = lax.axis_index("s")
    w = cc * NS + s
    pltpu.sync_copy(didx_hbm.at[w], didx_v)
    zero16 = jnp.zeros((16,), jnp.float32)
    ones16 = jnp.ones((16,), jnp.float32)

    def _zero(i, carry):
        dacc_v[pl.ds(i * 16, 16)] = zero16
        return carry

    lax.fori_loop(0, NPAD // 16, _zero, 0)

    def _count(i, carry):
        idx = didx_v[pl.ds(i * 16, 16)]
        plsc.addupdate_scatter(dacc_v, [idx], ones16)
        return carry

    lax.fori_loop(0, EPW // 16, _count, 0)
    pltpu.sync_copy(dacc_v, degp_hbm.at[w])


def _make_scatter(d, params):
    @functools.partial(
        pl.kernel,
        mesh=_MESH,
        out_type=jax.ShapeDtypeStruct((NC, NPAD, d), jnp.float32),
        scratch_types=[
            pltpu.VMEM((NCH, CH), jnp.int32),
            pltpu.VMEM((NCH, CH), jnp.int32),
            pltpu.VMEM((CH, d), jnp.float32),
            pltpu.VMEM_SHARED((NPAD, d), jnp.float32),
            pltpu.SemaphoreType.DMA,
        ],
        compiler_params=params,
    )
    def _scatter(p_hbm, sidx_hbm, didx_hbm, out_hbm, sidx_v, didx_v, rb, acc_sh, sem):
        cc = lax.axis_index("c")
        s = lax.axis_index("s")
        w = cc * NS + s
        pltpu.sync_copy(sidx_hbm.at[w], sidx_v)
        pltpu.sync_copy(didx_hbm.at[w], didx_v)

        zero16 = jnp.zeros((16,), jnp.float32)

        def _zero(r, carry):
            for k in range(d // 16):
                rb[r, pl.ds(k * 16, 16)] = zero16
            return carry

        lax.fori_loop(0, CH, _zero, 0)
        base = s * ROWS_PER_TILE
        for j in range(ROWS_PER_TILE // CH):
            pltpu.sync_copy(rb, acc_sh.at[pl.ds(base + j * CH, CH)])
        plsc.subcore_barrier()

        def _edge_chunk(c, carry):
            pltpu.async_copy(p_hbm.at[sidx_v.at[c]], rb, sem).wait()
            pltpu.sync_copy(rb, acc_sh.at[didx_v.at[c]], add=True)
            return carry

        lax.fori_loop(0, NCH, _edge_chunk, 0)
        plsc.subcore_barrier()
        for j in range(ROWS_PER_TILE // CH):
            pltpu.sync_copy(
                acc_sh.at[pl.ds(base + j * CH, CH)],
                out_hbm.at[cc, pl.ds(base + j * CH, CH)],
            )

    return _scatter


_scatter128 = _make_scatter(HIDDEN_K, _SC_PARAMS)
_scatter64 = _make_scatter(D_OUT_PAD, _SC_PARAMS_LINEAR)


def _dinv_from(degp):
    deg = jnp.sum(degp, axis=0) + 1.0
    return lax.rsqrt(deg)


def _scale_rows(v, dinv):
    d = v.shape[-1]
    v3 = v.reshape(BN // 128, 128, d)
    return (v3 * dinv[:, :, None]).reshape(BN, d)


def _tc1_body(x_ref, degp_ref, w_ref, o_ref):
    dinv = _dinv_from(degp_ref[...])
    h = jnp.dot(x_ref[...], w_ref[...], preferred_element_type=jnp.float32)
    o_ref[...] = _scale_rows(h, dinv)


def _tc_mid_body(s_ref, p_ref, degp_ref, b_ref, w_ref, o_ref):
    dinv = _dinv_from(degp_ref[...])
    agg = _scale_rows(s_ref[0] + s_ref[1] + p_ref[...], dinv)
    h = jnp.maximum(agg + b_ref[...], 0.0)
    o = jnp.dot(h, w_ref[...], preferred_element_type=jnp.float32)
    o_ref[...] = _scale_rows(o, dinv)


def _tc4_body(s_ref, p_ref, degp_ref, b_ref, o_ref):
    dinv = _dinv_from(degp_ref[...])
    agg = _scale_rows(s_ref[0] + s_ref[1] + p_ref[...], dinv)
    o_ref[...] = jnp.maximum(agg + b_ref[...], 0.0)


def _degp_spec():
    return pl.BlockSpec((NW, BN // 128, 128), lambda i: (0, i, 0))


def _tc1(x_pad, degp3, w):
    return pl.pallas_call(
        _tc1_body,
        grid=(GRID,),
        in_specs=[
            pl.BlockSpec((BN, D_FEAT_K), lambda i: (i, 0)),
            _degp_spec(),
            pl.BlockSpec((D_FEAT_K, HIDDEN_K), lambda i: (0, 0)),
        ],
        out_specs=pl.BlockSpec((BN, HIDDEN_K), lambda i: (i, 0)),
        out_shape=jax.ShapeDtypeStruct((NPAD, HIDDEN_K), jnp.float32),
    )(x_pad, degp3, w)


def _tc_mid(s, p, degp3, b, w, d_out):
    return pl.pallas_call(
        _tc_mid_body,
        grid=(GRID,),
        in_specs=[
            pl.BlockSpec((NC, BN, HIDDEN_K), lambda i: (0, i, 0)),
            pl.BlockSpec((BN, HIDDEN_K), lambda i: (i, 0)),
            _degp_spec(),
            pl.BlockSpec((1, HIDDEN_K), lambda i: (0, 0)),
            pl.BlockSpec((HIDDEN_K, d_out), lambda i: (0, 0)),
        ],
        out_specs=pl.BlockSpec((BN, d_out), lambda i: (i, 0)),
        out_shape=jax.ShapeDtypeStruct((NPAD, d_out), jnp.float32),
    )(s, p, degp3, b, w)


def _tc4(s, p, degp3, b):
    return pl.pallas_call(
        _tc4_body,
        grid=(GRID,),
        in_specs=[
            pl.BlockSpec((NC, BN, D_OUT_PAD), lambda i: (0, i, 0)),
            pl.BlockSpec((BN, D_OUT_PAD), lambda i: (i, 0)),
            _degp_spec(),
            pl.BlockSpec((1, D_OUT_PAD), lambda i: (0, 0)),
        ],
        out_specs=pl.BlockSpec((BN, D_OUT_PAD), lambda i: (i, 0)),
        out_shape=jax.ShapeDtypeStruct((NPAD, D_OUT_PAD), jnp.float32),
    )(s, p, degp3, b)


def kernel(x, edge_index, W_in, b_in, W_h, b_h, W_out, b_out):
    src = edge_index[0].astype(jnp.int32)
    dst = edge_index[1].astype(jnp.int32)
    pad_e = EPAD - N_EDGES_K
    src_p = jnp.concatenate(
        [src, jnp.zeros((pad_e,), jnp.int32)]).reshape(NW, NCH, CH)
    dst_p = jnp.concatenate(
        [dst, jnp.full((pad_e,), N_NODES_K, jnp.int32)]).reshape(NW, NCH, CH)
    dst_flat = dst_p.reshape(NW, EPW)

    x_pad = jnp.pad(x, ((0, NPAD - N_NODES_K), (0, 0)))
    wo_pad = jnp.pad(W_out, ((0, 0), (0, D_OUT_PAD - N_CLASSES_K)))
    bo_pad = jnp.pad(b_out, ((0, D_OUT_PAD - N_CLASSES_K),)).reshape(1, D_OUT_PAD)

    degp = _deg_kernel(dst_flat)
    degp3 = degp.reshape(NW, NPAD // 128, 128)

    p1 = _tc1(x_pad, degp3, W_in)
    s1 = _scatter128(p1, src_p, dst_p)
    p2 = _tc_mid(s1, p1, degp3, b_in.reshape(1, HIDDEN_K), W_h, HIDDEN_K)
    s2 = _scatter128(p2, src_p, dst_p)
    p3 = _tc_mid(s2, p2, degp3, b_h.reshape(1, HIDDEN_K), wo_pad, D_OUT_PAD)
    s3 = _scatter64(p3, src_p, dst_p)
    out = _tc4(s3, p3, degp3, bo_pad)
    return out[:N_NODES_K, :N_CLASSES_K]


# speedup vs baseline: 15.0574x; 1.1914x over previous
import functools

import jax
import jax.numpy as jnp
from jax import lax
from jax.experimental import pallas as pl
from jax.experimental.pallas import tpu as pltpu
from jax.experimental.pallas import tpu_sc as plsc

N_NODES_K = 10000
D_FEAT_K = 128
HIDDEN_K = 128
N_CLASSES_K = 40
N_EDGES_K = 320000

_info = plsc.get_sparse_core_info()
NC = _info.num_cores
NS = _info.num_subcores
NW = NC * NS

CH = 128
NCH = -(-N_EDGES_K // (NW * CH))
EPW = NCH * CH
EPAD = NW * EPW

NPAD = 10240
ROWS_PER_TILE = NPAD // NS
BN = 1024
GRID = NPAD // BN

D_OUT_PAD = 64

_MESH = plsc.VectorSubcoreMesh(core_axis_name="c", subcore_axis_name="s")
_SC_PARAMS = pltpu.CompilerParams(needs_layout_passes=False)
_SC_PARAMS_LINEAR = pltpu.CompilerParams(
    needs_layout_passes=False, use_tc_tiling_on_sc=False)


@functools.partial(
    pl.kernel,
    mesh=_MESH,
    out_type=jax.ShapeDtypeStruct((NW, NPAD), jnp.float32),
    scratch_types=[
        pltpu.VMEM((EPW,), jnp.int32),
        pltpu.VMEM((NPAD,), jnp.float32),
    ],
    compiler_params=_SC_PARAMS,
)
def _deg_kernel(didx_hbm, degp_hbm, didx_v, dacc_v):
    cc = lax.axis_index("c")
    s = lax.axis_index("s")
    w = cc * NS + s
    pltpu.sync_copy(didx_hbm.at[w], didx_v)
    zero16 = jnp.zeros((16,), jnp.float32)
    ones16 = jnp.ones((16,), jnp.float32)

    def _zero(i, carry):
        dacc_v[pl.ds(i * 16, 16)] = zero16
        return carry

    lax.fori_loop(0, NPAD // 16, _zero, 0)

    def _count(i, carry):
        idx = didx_v[pl.ds(i * 16, 16)]
        plsc.addupdate_scatter(dacc_v, [idx], ones16)
        return carry

    lax.fori_loop(0, EPW // 16, _count, 0)
    pltpu.sync_copy(dacc_v, degp_hbm.at[w])


def _make_scatter(d, params):
    @functools.partial(
        pl.kernel,
        mesh=_MESH,
        out_type=jax.ShapeDtypeStruct((NC, NPAD, d), jnp.float32),
        scratch_types=[
            pltpu.VMEM((1, CH), jnp.int32),
            pltpu.VMEM((1, CH), jnp.int32),
            pltpu.VMEM((NCH, CH), jnp.int32),
            pltpu.VMEM((CH, d), jnp.float32),
            pltpu.VMEM((CH, d), jnp.float32),
            pltpu.VMEM_SHARED((NPAD, d), jnp.float32),
            pltpu.SemaphoreType.DMA,
            pltpu.SemaphoreType.DMA,
            pltpu.SemaphoreType.DMA,
            pltpu.SemaphoreType.DMA,
        ],
        compiler_params=params,
    )
    def _scatter(p_hbm, sidx_hbm, didx_hbm, out_hbm, ib0, ib1, didx_v, rb0,
                 rb1, acc_sh, semi0, semi1, semg0, semg1):
        cc = lax.axis_index("c")
        s = lax.axis_index("s")
        w = cc * NS + s
        pltpu.sync_copy(didx_hbm.at[w], didx_v)

        zero16 = jnp.zeros((16,), jnp.float32)

        def _zero(r, carry):
            for k in range(d // 16):
                rb0[r, pl.ds(k * 16, 16)] = zero16
            return carry

        lax.fori_loop(0, CH, _zero, 0)
        base = s * ROWS_PER_TILE
        for j in range(ROWS_PER_TILE // CH):
            pltpu.sync_copy(rb0, acc_sh.at[pl.ds(base + j * CH, CH)])

        pltpu.sync_copy(sidx_hbm.at[w, pl.ds(0, 1)], ib0)
        pltpu.async_copy(p_hbm.at[ib0.at[0]], rb0, semg0)
        pltpu.async_copy(sidx_hbm.at[w, pl.ds(1, 1)], ib1, semi1)
        plsc.subcore_barrier()

        def _edge_pair(j, carry):
            c0 = j * 2
            pltpu.make_async_copy(sidx_hbm.at[w, pl.ds(c0 + 1, 1)], ib1, semi1).wait()
            pltpu.make_async_copy(p_hbm.at[ib0.at[0]], rb0, semg0).wait()
            pltpu.async_copy(p_hbm.at[ib1.at[0]], rb1, semg1)
            pltpu.async_copy(sidx_hbm.at[w, pl.ds(c0 + 2, 1)], ib0, semi0)
            pltpu.sync_copy(rb0, acc_sh.at[didx_v.at[c0]], add=True)
            pltpu.make_async_copy(sidx_hbm.at[w, pl.ds(c0 + 2, 1)], ib0, semi0).wait()
            pltpu.make_async_copy(p_hbm.at[ib1.at[0]], rb1, semg1).wait()
            pltpu.async_copy(p_hbm.at[ib0.at[0]], rb0, semg0)
            pltpu.async_copy(sidx_hbm.at[w, pl.ds(c0 + 3, 1)], ib1, semi1)
            pltpu.sync_copy(rb1, acc_sh.at[didx_v.at[c0 + 1]], add=True)
            return carry

        lax.fori_loop(0, (NCH - 1) // 2, _edge_pair, 0)
        pltpu.make_async_copy(p_hbm.at[ib0.at[0]], rb0, semg0).wait()
        pltpu.make_async_copy(sidx_hbm.at[w, pl.ds(NCH, 1)], ib1, semi1).wait()
        pltpu.sync_copy(rb0, acc_sh.at[didx_v.at[NCH - 1]], add=True)
        plsc.subcore_barrier()
        for j in range(ROWS_PER_TILE // CH):
            pltpu.sync_copy(
                acc_sh.at[pl.ds(base + j * CH, CH)],
                out_hbm.at[cc, pl.ds(base + j * CH, CH)],
            )

    return _scatter


_scatter128 = _make_scatter(HIDDEN_K, _SC_PARAMS)
_scatter64 = _make_scatter(D_OUT_PAD, _SC_PARAMS_LINEAR)


def _dinv_from(degp):
    deg = jnp.sum(degp, axis=0) + 1.0
    return lax.rsqrt(deg)


def _scale_rows(v, dinv):
    d = v.shape[-1]
    v3 = v.reshape(BN // 128, 128, d)
    return (v3 * dinv[:, :, None]).reshape(BN, d)


def _tc1_body(x_ref, degp_ref, w_ref, o_ref):
    dinv = _dinv_from(degp_ref[...])
    h = jnp.dot(x_ref[...], w_ref[...], preferred_element_type=jnp.float32)
    o_ref[...] = _scale_rows(h, dinv)


def _tc_mid_body(s_ref, p_ref, degp_ref, b_ref, w_ref, o_ref):
    dinv = _dinv_from(degp_ref[...])
    agg = _scale_rows(s_ref[0] + s_ref[1] + p_ref[...], dinv)
    h = jnp.maximum(agg + b_ref[...], 0.0)
    o = jnp.dot(h, w_ref[...], preferred_element_type=jnp.float32)
    o_ref[...] = _scale_rows(o, dinv)


def _tc4_body(s_ref, p_ref, degp_ref, b_ref, o_ref):
    dinv = _dinv_from(degp_ref[...])
    agg = _scale_rows(s_ref[0] + s_ref[1] + p_ref[...], dinv)
    o_ref[...] = jnp.maximum(agg + b_ref[...], 0.0)


def _degp_spec():
    return pl.BlockSpec((NW, BN // 128, 128), lambda i: (0, i, 0))


def _tc1(x_pad, degp3, w):
    return pl.pallas_call(
        _tc1_body,
        grid=(GRID,),
        in_specs=[
            pl.BlockSpec((BN, D_FEAT_K), lambda i: (i, 0)),
            _degp_spec(),
            pl.BlockSpec((D_FEAT_K, HIDDEN_K), lambda i: (0, 0)),
        ],
        out_specs=pl.BlockSpec((BN, HIDDEN_K), lambda i: (i, 0)),
        out_shape=jax.ShapeDtypeStruct((NPAD, HIDDEN_K), jnp.float32),
    )(x_pad, degp3, w)


def _tc_mid(s, p, degp3, b, w, d_out):
    return pl.pallas_call(
        _tc_mid_body,
        grid=(GRID,),
        in_specs=[
            pl.BlockSpec((NC, BN, HIDDEN_K), lambda i: (0, i, 0)),
            pl.BlockSpec((BN, HIDDEN_K), lambda i: (i, 0)),
            _degp_spec(),
            pl.BlockSpec((1, HIDDEN_K), lambda i: (0, 0)),
            pl.BlockSpec((HIDDEN_K, d_out), lambda i: (0, 0)),
        ],
        out_specs=pl.BlockSpec((BN, d_out), lambda i: (i, 0)),
        out_shape=jax.ShapeDtypeStruct((NPAD, d_out), jnp.float32),
    )(s, p, degp3, b, w)


def _tc4(s, p, degp3, b):
    return pl.pallas_call(
        _tc4_body,
        grid=(GRID,),
        in_specs=[
            pl.BlockSpec((NC, BN, D_OUT_PAD), lambda i: (0, i, 0)),
            pl.BlockSpec((BN, D_OUT_PAD), lambda i: (i, 0)),
            _degp_spec(),
            pl.BlockSpec((1, D_OUT_PAD), lambda i: (0, 0)),
        ],
        out_specs=pl.BlockSpec((BN, D_OUT_PAD), lambda i: (i, 0)),
        out_shape=jax.ShapeDtypeStruct((NPAD, D_OUT_PAD), jnp.float32),
    )(s, p, degp3, b)


def kernel(x, edge_index, W_in, b_in, W_h, b_h, W_out, b_out):
    src = edge_index[0].astype(jnp.int32)
    dst = edge_index[1].astype(jnp.int32)
    pad_e = EPAD - N_EDGES_K
    src_p = jnp.concatenate(
        [src, jnp.zeros((pad_e,), jnp.int32)]).reshape(NW, NCH, CH)
    src_p = jnp.pad(src_p, ((0, 0), (0, 1), (0, 0)))
    dst_p = jnp.concatenate(
        [dst, jnp.full((pad_e,), N_NODES_K, jnp.int32)]).reshape(NW, NCH, CH)
    dst_flat = dst_p.reshape(NW, EPW)

    x_pad = jnp.pad(x, ((0, NPAD - N_NODES_K), (0, 0)))
    wo_pad = jnp.pad(W_out, ((0, 0), (0, D_OUT_PAD - N_CLASSES_K)))
    bo_pad = jnp.pad(b_out, ((0, D_OUT_PAD - N_CLASSES_K),)).reshape(1, D_OUT_PAD)

    degp = _deg_kernel(dst_flat)
    degp3 = degp.reshape(NW, NPAD // 128, 128)

    p1 = _tc1(x_pad, degp3, W_in)
    s1 = _scatter128(p1, src_p, dst_p)
    p2 = _tc_mid(s1, p1, degp3, b_in.reshape(1, HIDDEN_K), W_h, HIDDEN_K)
    s2 = _scatter128(p2, src_p, dst_p)
    p3 = _tc_mid(s2, p2, degp3, b_h.reshape(1, HIDDEN_K), wo_pad, D_OUT_PAD)
    s3 = _scatter64(p3, src_p, dst_p)
    out = _tc4(s3, p3, degp3, bo_pad)
    return out[:N_NODES_K, :N_CLASSES_K]
